# baseline (speedup 1.0000x reference)
import functools
import math

import jax
import jax.numpy as jnp
from jax import lax
from jax.experimental import pallas as pl
from jax.experimental.pallas import tpu as pltpu
from jax.experimental.pallas import tpu_sc as plsc

D_MODEL = 2048
SCALE = math.sqrt(D_MODEL)
LANES = 16
NUM_CORES = 2
NUM_SUBCORES = 16
NUM_WORKERS = NUM_CORES * NUM_SUBCORES

CHUNK = 16


def _build_kernel(B):
    b_per_w = B // NUM_WORKERS
    n_chunks = b_per_w // CHUNK
    mesh = plsc.VectorSubcoreMesh(core_axis_name="c", subcore_axis_name="s")

    @functools.partial(
        pl.kernel,
        mesh=mesh,
        out_type=jax.ShapeDtypeStruct((B, D_MODEL), jnp.float32),
        scratch_types=[
            pltpu.VMEM((b_per_w,), jnp.int32),
            pltpu.VMEM((CHUNK, D_MODEL), jnp.float32),
            pltpu.SemaphoreType.DMA,
        ],
    )
    def k(lut_hbm, idx_hbm, out_hbm, idx_v, rows_v, sem):
        wid = lax.axis_index("s") * NUM_CORES + lax.axis_index("c")
        base = wid * b_per_w
        pltpu.sync_copy(idx_hbm.at[pl.ds(base, b_per_w)], idx_v)

        @pl.loop(0, n_chunks)
        def _chunk(cg):
            pltpu.async_copy(
                lut_hbm.at[idx_v.at[pl.ds(cg * CHUNK, CHUNK)]], rows_v, sem
            ).wait()

            @pl.loop(0, CHUNK)
            def _row(i):
                for j in range(D_MODEL // LANES):
                    sl = pl.ds(j * LANES, LANES)
                    rows_v[i, sl] = rows_v[i, sl] * SCALE

            pltpu.sync_copy(rows_v, out_hbm.at[pl.ds(base + cg * CHUNK, CHUNK)])

    return k


def kernel(x, lut):
    B = x.size
    idx = x.reshape(B).astype(jnp.int32)
    out = _build_kernel(B)(lut, idx)
    return out.reshape(x.shape + (D_MODEL,))


# speedup vs baseline: 1.5274x; 1.5274x over previous
import functools
import math

import jax
import jax.numpy as jnp
from jax import lax
from jax.experimental import pallas as pl
from jax.experimental.pallas import tpu as pltpu
from jax.experimental.pallas import tpu_sc as plsc

D_MODEL = 2048
SCALE = math.sqrt(D_MODEL)
LANES = 16
NUM_CORES = 2
NUM_SUBCORES = 16
NUM_WORKERS = NUM_CORES * NUM_SUBCORES

CHUNK = 16


def _build_kernel(B):
    b_per_w = B // NUM_WORKERS
    n_chunks = b_per_w // CHUNK
    mesh = plsc.VectorSubcoreMesh(core_axis_name="c", subcore_axis_name="s")

    @functools.partial(
        pl.kernel,
        mesh=mesh,
        out_type=jax.ShapeDtypeStruct((B, D_MODEL), jnp.float32),
        scratch_types=[
            pltpu.VMEM((b_per_w,), jnp.int32),
            pltpu.VMEM((2, CHUNK, D_MODEL), jnp.float32),
            pltpu.SemaphoreType.DMA,
            pltpu.SemaphoreType.DMA,
        ],
    )
    def k(lut_hbm, idx_hbm, out_hbm, idx_v, rows_v, gsem, ssem):
        wid = lax.axis_index("s") * NUM_CORES + lax.axis_index("c")
        base = wid * b_per_w
        pltpu.sync_copy(idx_hbm.at[pl.ds(base, b_per_w)], idx_v)

        pltpu.async_copy(lut_hbm.at[idx_v.at[pl.ds(0, CHUNK)]], rows_v.at[0], gsem)

        @pl.loop(0, n_chunks, step=2)
        def _chunk(g):
            for b in range(2):
                cg = g + b
                o = 1 - b

                @pl.when(cg >= 1)
                def _():
                    pltpu.make_async_copy(
                        rows_v.at[o], out_hbm.at[pl.ds(base, CHUNK)], ssem
                    ).wait()

                @pl.when(cg + 1 < n_chunks)
                def _():
                    pltpu.async_copy(
                        lut_hbm.at[idx_v.at[pl.ds((cg + 1) * CHUNK, CHUNK)]],
                        rows_v.at[o],
                        gsem,
                    )

                pltpu.make_async_copy(
                    lut_hbm.at[pl.ds(0, CHUNK)], rows_v.at[b], gsem
                ).wait()

                @pl.loop(0, CHUNK)
                def _row(i):
                    for j in range(D_MODEL // LANES):
                        sl = pl.ds(j * LANES, LANES)
                        rows_v[b, i, sl] = rows_v[b, i, sl] * SCALE

                pltpu.async_copy(
                    rows_v.at[b], out_hbm.at[pl.ds(base + cg * CHUNK, CHUNK)], ssem
                )

        pltpu.make_async_copy(
            rows_v.at[1], out_hbm.at[pl.ds(base, CHUNK)], ssem
        ).wait()

    return k


def kernel(x, lut):
    B = x.size
    idx = x.reshape(B).astype(jnp.int32)
    out = _build_kernel(B)(lut, idx)
    return out.reshape(x.shape + (D_MODEL,))


# speedup vs baseline: 1.8031x; 1.1805x over previous
import functools
import math

import jax
import jax.numpy as jnp
from jax import lax
from jax.experimental import pallas as pl
from jax.experimental.pallas import tpu as pltpu
from jax.experimental.pallas import tpu_sc as plsc

D_MODEL = 2048
SCALE = math.sqrt(D_MODEL)
LANES = 16
NUM_CORES = 2
NUM_SUBCORES = 16
NUM_WORKERS = NUM_CORES * NUM_SUBCORES

CHUNK = 8
NBUF = 4


def _build_kernel(B):
    b_per_w = B // NUM_WORKERS
    n_chunks = b_per_w // CHUNK
    mesh = plsc.VectorSubcoreMesh(core_axis_name="c", subcore_axis_name="s")

    @functools.partial(
        pl.kernel,
        mesh=mesh,
        out_type=jax.ShapeDtypeStruct((B, D_MODEL), jnp.float32),
        scratch_types=[
            pltpu.VMEM((b_per_w,), jnp.int32),
            pltpu.VMEM((NBUF, CHUNK, D_MODEL), jnp.float32),
            pltpu.SemaphoreType.DMA,
            pltpu.SemaphoreType.DMA,
        ],
    )
    def k(lut_hbm, idx_hbm, out_hbm, idx_v, rows_v, gsem, ssem):
        wid = lax.axis_index("s") * NUM_CORES + lax.axis_index("c")
        base = wid * b_per_w
        pltpu.sync_copy(idx_hbm.at[pl.ds(base, b_per_w)], idx_v)

        def start_gather(cg, buf):
            pltpu.async_copy(
                lut_hbm.at[idx_v.at[pl.ds(cg * CHUNK, CHUNK)]],
                rows_v.at[buf],
                gsem,
            )

        start_gather(0, 0)
        start_gather(1, 1)

        @pl.loop(0, n_chunks, step=NBUF)
        def _chunk(g):
            for b in range(NBUF):
                cg = g + b
                nb = (b + 2) % NBUF

                @pl.when(cg >= 2)
                def _():
                    pltpu.make_async_copy(
                        rows_v.at[nb], out_hbm.at[pl.ds(base, CHUNK)], ssem
                    ).wait()

                @pl.when(cg + 2 < n_chunks)
                def _():
                    start_gather(cg + 2, nb)

                pltpu.make_async_copy(
                    lut_hbm.at[pl.ds(0, CHUNK)], rows_v.at[b], gsem
                ).wait()

                @pl.loop(0, CHUNK)
                def _row(i):
                    for j in range(D_MODEL // LANES):
                        sl = pl.ds(j * LANES, LANES)
                        rows_v[b, i, sl] = rows_v[b, i, sl] * SCALE

                pltpu.async_copy(
                    rows_v.at[b], out_hbm.at[pl.ds(base + cg * CHUNK, CHUNK)], ssem
                )

        for tail in (n_chunks - 2, n_chunks - 1):
            pltpu.make_async_copy(
                rows_v.at[tail % NBUF], out_hbm.at[pl.ds(base, CHUNK)], ssem
            ).wait()

    return k


def kernel(x, lut):
    B = x.size
    idx = x.reshape(B).astype(jnp.int32)
    out = _build_kernel(B)(lut, idx)
    return out.reshape(x.shape + (D_MODEL,))
